# baseline (speedup 1.0000x reference)
import functools

import jax
import jax.numpy as jnp
from jax import lax
from jax.experimental import pallas as pl
from jax.experimental.pallas import tpu as pltpu
from jax.experimental.pallas import tpu_sc as plsc


_ROWS = 16384
_DIM = 128
_NC = 2
_NS = 16
_NW = _NC * _NS
_RPW = _ROWS // _NW


_mesh = plsc.VectorSubcoreMesh(core_axis_name="c", subcore_axis_name="s")


@functools.partial(
    pl.kernel,
    mesh=_mesh,
    out_type=jax.ShapeDtypeStruct((_ROWS, _DIM), jnp.float32),
    scratch_types=[pltpu.VMEM((_RPW, _DIM), jnp.float32)],
)
def _sc_copy(q_hbm, out_hbm, buf):
    wid = lax.axis_index("s") * _NC + lax.axis_index("c")
    base = wid * _RPW
    pltpu.sync_copy(q_hbm.at[pl.ds(base, _RPW)], buf)
    pltpu.sync_copy(buf, out_hbm.at[pl.ds(base, _RPW)])


def _mask_kernel(mask_ref):
    mask_ref[...] = jnp.ones(mask_ref.shape, dtype=jnp.bool_)


def kernel(query, relation_weight):
    out = _sc_copy(query)
    mask = pl.pallas_call(
        _mask_kernel,
        out_shape=jax.ShapeDtypeStruct((_DIM, _DIM), jnp.bool_),
    )()
    return (out.reshape(_ROWS, 1, _DIM), mask.reshape(_ROWS, 1))


# speedup vs baseline: 3.0339x; 3.0339x over previous
import jax
import jax.numpy as jnp
from jax.experimental import pallas as pl
from jax.experimental.pallas import tpu as pltpu


_ROWS = 16384
_DIM = 128
_NCHUNK = 4
_CHUNK = _ROWS // _NCHUNK


def _copy_kernel(q_hbm, out_hbm, mask_ref, b0, b1, b2, b3, in_sem, out_sem):
    bufs = (b0, b1, b2, b3)
    for i in range(_NCHUNK):
        pltpu.make_async_copy(
            q_hbm.at[pl.ds(i * _CHUNK, _CHUNK)], bufs[i], in_sem.at[i]
        ).start()
    mask_ref[...] = jnp.ones(mask_ref.shape, dtype=jnp.bool_)
    for i in range(_NCHUNK):
        pltpu.make_async_copy(
            q_hbm.at[pl.ds(i * _CHUNK, _CHUNK)], bufs[i], in_sem.at[i]
        ).wait()
        pltpu.make_async_copy(
            bufs[i], out_hbm.at[pl.ds(i * _CHUNK, _CHUNK)], out_sem.at[i]
        ).start()
    for i in range(_NCHUNK):
        pltpu.make_async_copy(
            bufs[i], out_hbm.at[pl.ds(i * _CHUNK, _CHUNK)], out_sem.at[i]
        ).wait()


def kernel(query, relation_weight):
    out, mask = pl.pallas_call(
        _copy_kernel,
        in_specs=[pl.BlockSpec(memory_space=pl.ANY)],
        out_specs=[
            pl.BlockSpec(memory_space=pl.ANY),
            pl.BlockSpec(memory_space=pltpu.MemorySpace.VMEM),
        ],
        out_shape=[
            jax.ShapeDtypeStruct((_ROWS, _DIM), jnp.float32),
            jax.ShapeDtypeStruct((_DIM, _DIM), jnp.bool_),
        ],
        scratch_shapes=[
            pltpu.VMEM((_CHUNK, _DIM), jnp.float32),
            pltpu.VMEM((_CHUNK, _DIM), jnp.float32),
            pltpu.VMEM((_CHUNK, _DIM), jnp.float32),
            pltpu.VMEM((_CHUNK, _DIM), jnp.float32),
            pltpu.SemaphoreType.DMA((_NCHUNK,)),
            pltpu.SemaphoreType.DMA((_NCHUNK,)),
        ],
    )(query)
    return (out.reshape(_ROWS, 1, _DIM), mask.reshape(_ROWS, 1))
